# baseline (speedup 1.0000x reference)
import functools

import jax
import jax.numpy as jnp
from jax import lax
from jax.experimental import pallas as pl
from jax.experimental.pallas import tpu as pltpu
from jax.experimental.pallas import tpu_sc as plsc

NC = 2
NS = 16
NW = NC * NS


def _gemm_body(x_ref, w_ref, o_ref):
    o_ref[0] = jnp.dot(x_ref[...], w_ref[0],
                       preferred_element_type=jnp.float32)


def _relation_gemm(x, weight, bn):
    n, d_in = x.shape
    r, _, d_out = weight.shape
    return pl.pallas_call(
        _gemm_body,
        grid=(r, n // bn),
        in_specs=[
            pl.BlockSpec((bn, d_in), lambda i, j: (j, 0)),
            pl.BlockSpec((1, d_in, d_out), lambda i, j: (i, 0, 0)),
        ],
        out_specs=pl.BlockSpec((1, bn, d_out), lambda i, j: (i, j, 0)),
        out_shape=jax.ShapeDtypeStruct((r, n, d_out), jnp.float32),
    )(x, weight)


def _gid_body(n_nodes, s_ref, r_ref, o_ref):
    o_ref[...] = r_ref[...] * n_nodes + s_ref[...]


def _edge_gid(src, rel, n_nodes):
    e = src.shape[0]
    s2 = src.reshape(e // 128, 128)
    r2 = rel.reshape(e // 128, 128)
    out = pl.pallas_call(
        functools.partial(_gid_body, n_nodes),
        out_shape=jax.ShapeDtypeStruct(s2.shape, jnp.int32),
    )(s2, r2)
    return out.reshape(e)


def _add_body(p_ref, o_ref):
    o_ref[...] = p_ref[0] + p_ref[1]


def _pair_add(p, bn):
    _, n, d = p.shape
    return pl.pallas_call(
        _add_body,
        grid=(n // bn,),
        in_specs=[pl.BlockSpec((2, bn, d), lambda i: (0, i, 0))],
        out_specs=pl.BlockSpec((bn, d), lambda i: (i, 0)),
        out_shape=jax.ShapeDtypeStruct((n, d), jnp.float32),
    )(p)


def _make_sc_scatter(n_nodes, d, n_edges):
    ept = n_edges // NW
    b = 64
    nch = -(-ept // b)
    nch += (2 - nch) % 4
    ept_p = nch * b
    rpt_a = (n_nodes // NS) & ~7
    rpt_z = n_nodes - rpt_a * (NS - 1)
    assert ept * NW == n_edges and nch % 4 == 2 and nch >= 10
    assert b % 8 == 0 and ept_p % 8 == 0 and rpt_a % 8 == 0

    mesh = plsc.VectorSubcoreMesh(core_axis_name="c", subcore_axis_name="s",
                                  num_cores=NC, num_subcores=NS)

    @functools.partial(
        pl.kernel,
        out_type=jax.ShapeDtypeStruct((NC, n_nodes, d), jnp.float32),
        mesh=mesh,
        scratch_types=[
            pltpu.VMEM((ept_p,), jnp.int32),
            pltpu.VMEM((4, b), jnp.int32),
            pltpu.VMEM((4, b, d), jnp.float32),
            pltpu.VMEM_SHARED((n_nodes + 8, d), jnp.float32),
            pltpu.SemaphoreType.DMA,
            pltpu.SemaphoreType.DMA,
            (pltpu.SemaphoreType.DMA,) * 4,
            (pltpu.SemaphoreType.DMA,) * 4,
        ],
    )
    def sc_scatter(y_hbm, g_hbm, dst_hbm, zeros_hbm, out_hbm,
                   gid, didr, rows, h_sh, sem_i, sem_d, sg, ss):
        cid = lax.axis_index("c")
        sid = lax.axis_index("s")
        wid = cid * NS + sid
        base = wid * ept_p

        cp_g = pltpu.async_copy(g_hbm.at[pl.ds(base, ept_p)], gid, sem_i)

        def fire_d(c, slot):
            pltpu.async_copy(dst_hbm.at[pl.ds(base + c * b, b)],
                             didr.at[slot], sem_d)

        def wait_d():
            pltpu.make_async_copy(dst_hbm.at[pl.ds(base, b)],
                                  didr.at[0], sem_d).wait()

        @pl.when(sid < NS - 1)
        def _():
            pltpu.sync_copy(zeros_hbm.at[pl.ds(sid * rpt_a, rpt_a)],
                            h_sh.at[pl.ds(sid * rpt_a, rpt_a)])

        @pl.when(sid == NS - 1)
        def _():
            pltpu.sync_copy(zeros_hbm.at[pl.ds(rpt_a * (NS - 1), rpt_z)],
                            h_sh.at[pl.ds(rpt_a * (NS - 1), rpt_z)])

        cp_g.wait()

        plsc.subcore_barrier()

        def fire_g(c, slot):
            pltpu.async_copy(y_hbm.at[gid.at[pl.ds(c * b, b)]],
                             rows.at[slot], sg[slot])

        def wait_g(c, slot):
            pltpu.make_async_copy(y_hbm.at[gid.at[pl.ds(c * b, b)]],
                                  rows.at[slot], sg[slot]).wait()

        def fire_s(c, slot):
            pltpu.async_copy(rows.at[slot], h_sh.at[didr.at[slot]],
                             ss[slot], add=True)

        def wait_s(slot):
            pltpu.make_async_copy(rows.at[slot], h_sh.at[didr.at[slot]],
                                  ss[slot]).wait()

        def visit(c, k, do_wait_s, do_fire_g):
            wait_g(c, k)
            wait_d()
            fire_s(c, k)
            if do_wait_s:
                wait_s((k + 2) % 4)
            if do_fire_g:
                fire_d(c + 2, (k + 2) % 4)
                fire_g(c + 2, (k + 2) % 4)

        fire_d(0, 0)
        fire_d(1, 1)
        fire_g(0, 0)
        fire_g(1, 1)
        for c in range(4):
            visit(c, c, do_wait_s=(c >= 2), do_fire_g=True)

        def step(t, carry):
            for k in range(4):
                visit(4 * t + k, k, do_wait_s=True, do_fire_g=True)
            return carry

        lax.fori_loop(1, (nch - 8) // 4 + 1, step, 0)
        for c in range(nch - 6, nch):
            visit(c, c % 4, do_wait_s=True, do_fire_g=(c + 2 < nch))
        wait_s((nch - 2) % 4)
        wait_s((nch - 1) % 4)

        plsc.subcore_barrier()

        @pl.when(sid < NS - 1)
        def _():
            pltpu.sync_copy(h_sh.at[pl.ds(sid * rpt_a, rpt_a)],
                            out_hbm.at[cid, pl.ds(sid * rpt_a, rpt_a)])

        @pl.when(sid == NS - 1)
        def _():
            pltpu.sync_copy(h_sh.at[pl.ds(rpt_a * (NS - 1), rpt_z)],
                            out_hbm.at[cid, pl.ds(rpt_a * (NS - 1), rpt_z)])

    return sc_scatter, ept, ept_p


def _pad_per_tile(a, ept, ept_p, fill):
    if ept_p == ept:
        return a
    pad = jnp.full((NW, ept_p - ept), fill, a.dtype)
    return jnp.concatenate([a.reshape(NW, ept), pad], axis=1).reshape(-1)


def kernel(x, edge_index, rel_type, weight):
    n, _ = x.shape
    r, _, d_out = weight.shape
    e = edge_index.shape[1]
    src = edge_index[0]
    dst = edge_index[1]
    y = _relation_gemm(x, weight, 1000).reshape(r * n, d_out)
    g = _edge_gid(src, rel_type, n)
    sc, ept, ept_p = _make_sc_scatter(n, d_out, e)
    g_p = _pad_per_tile(g, ept, ept_p, 0)
    dst_p = _pad_per_tile(dst, ept, ept_p, n)
    zeros = jnp.zeros((n, d_out), jnp.float32)
    partials = sc(y, g_p, dst_p, zeros)
    return _pair_add(partials, 1000)


# speedup vs baseline: 1.0545x; 1.0545x over previous
import functools

import jax
import jax.numpy as jnp
from jax import lax
from jax.experimental import pallas as pl
from jax.experimental.pallas import tpu as pltpu
from jax.experimental.pallas import tpu_sc as plsc

NC = 2
NS = 16
NW = NC * NS


def _gemm_gid_body(n_nodes, x_ref, w_ref, s_ref, r_ref, o_ref, g_ref):
    o_ref[0] = jnp.dot(x_ref[...], w_ref[0],
                       preferred_element_type=jnp.float32)
    g_ref[...] = r_ref[...] * n_nodes + s_ref[...]


def _relation_gemm_gid(x, weight, src, rel, bn):
    n, d_in = x.shape
    r, _, d_out = weight.shape
    e = src.shape[0]
    nb = r * (n // bn)
    eb = 8
    ew = e // (nb * eb)
    assert eb * ew * nb == e
    s2 = src.reshape(nb * eb, ew)
    r2 = rel.reshape(nb * eb, ew)
    return pl.pallas_call(
        functools.partial(_gemm_gid_body, n),
        grid=(r, n // bn),
        in_specs=[
            pl.BlockSpec((bn, d_in), lambda i, j: (j, 0)),
            pl.BlockSpec((1, d_in, d_out), lambda i, j: (i, 0, 0)),
            pl.BlockSpec((eb, ew), lambda i, j, _nbj=n // bn: (i * _nbj + j, 0)),
            pl.BlockSpec((eb, ew), lambda i, j, _nbj=n // bn: (i * _nbj + j, 0)),
        ],
        out_specs=[
            pl.BlockSpec((1, bn, d_out), lambda i, j: (i, j, 0)),
            pl.BlockSpec((eb, ew), lambda i, j, _nbj=n // bn: (i * _nbj + j, 0)),
        ],
        out_shape=[
            jax.ShapeDtypeStruct((r, n, d_out), jnp.float32),
            jax.ShapeDtypeStruct((nb * eb, ew), jnp.int32),
        ],
    )(x, weight, s2, r2)


def _add_body(p_ref, o_ref):
    o_ref[...] = p_ref[0] + p_ref[1]


def _pair_add(p, bn):
    _, n, d = p.shape
    return pl.pallas_call(
        _add_body,
        grid=(n // bn,),
        in_specs=[pl.BlockSpec((2, bn, d), lambda i: (0, i, 0))],
        out_specs=pl.BlockSpec((bn, d), lambda i: (i, 0)),
        out_shape=jax.ShapeDtypeStruct((n, d), jnp.float32),
    )(p)


def _make_sc_scatter(n_nodes, d, n_edges):
    ept = n_edges // NW
    b = 96
    nch = -(-ept // b)
    nch += (1 - nch) % 2
    ept_p = nch * b
    rpt_a = (n_nodes // NS) & ~7
    rpt_z = n_nodes - rpt_a * (NS - 1)
    assert ept * NW == n_edges and nch % 2 == 1 and nch >= 5
    assert b % 8 == 0 and ept_p % 8 == 0 and rpt_a % 8 == 0

    mesh = plsc.VectorSubcoreMesh(core_axis_name="c", subcore_axis_name="s",
                                  num_cores=NC, num_subcores=NS)

    @functools.partial(
        pl.kernel,
        out_type=jax.ShapeDtypeStruct((NC, n_nodes, d), jnp.float32),
        mesh=mesh,
        scratch_types=[
            pltpu.VMEM((ept_p,), jnp.int32),
            pltpu.VMEM((nch, b), jnp.int32),
            pltpu.VMEM((2, b, d), jnp.float32),
            pltpu.VMEM_SHARED((n_nodes + 8, d), jnp.float32),
            pltpu.SemaphoreType.DMA,
            pltpu.SemaphoreType.DMA,
            pltpu.SemaphoreType.DMA,
            pltpu.SemaphoreType.DMA,
        ],
    )
    def sc_scatter(y_hbm, g_hbm, dst_hbm, zeros_hbm, out_hbm,
                   gid, did2, rows, h_sh, sem_i, sem_d, sem_a, sem_b):
        cid = lax.axis_index("c")
        sid = lax.axis_index("s")
        wid = cid * NS + sid
        base = wid * ept_p

        cp_g = pltpu.async_copy(g_hbm.at[pl.ds(base, ept_p)], gid, sem_i)

        def fill_did(c, carry):
            pltpu.async_copy(dst_hbm.at[pl.ds(base + c * b, b)],
                             did2.at[c], sem_d)
            return carry

        lax.fori_loop(0, nch, fill_did, 0)

        @pl.when(sid < NS - 1)
        def _():
            pltpu.sync_copy(zeros_hbm.at[pl.ds(sid * rpt_a, rpt_a)],
                            h_sh.at[pl.ds(sid * rpt_a, rpt_a)])

        @pl.when(sid == NS - 1)
        def _():
            pltpu.sync_copy(zeros_hbm.at[pl.ds(rpt_a * (NS - 1), rpt_z)],
                            h_sh.at[pl.ds(rpt_a * (NS - 1), rpt_z)])

        cp_g.wait()

        def drain_did(c, carry):
            pltpu.make_async_copy(dst_hbm.at[pl.ds(base, b)],
                                  did2.at[0], sem_d).wait()
            return carry

        lax.fori_loop(0, nch, drain_did, 0)

        plsc.subcore_barrier()

        pltpu.async_copy(y_hbm.at[gid.at[pl.ds(0, b)]], rows.at[0], sem_a)

        def step(t, carry):
            c0 = 2 * t
            c1 = 2 * t + 1
            pltpu.make_async_copy(y_hbm.at[gid.at[pl.ds(c0 * b, b)]],
                                  rows.at[0], sem_a).wait()
            pltpu.async_copy(y_hbm.at[gid.at[pl.ds(c1 * b, b)]],
                             rows.at[1], sem_b)
            pltpu.sync_copy(rows.at[0], h_sh.at[did2.at[c0]], add=True)
            pltpu.make_async_copy(y_hbm.at[gid.at[pl.ds(c1 * b, b)]],
                                  rows.at[1], sem_b).wait()
            pltpu.async_copy(y_hbm.at[gid.at[pl.ds((c1 + 1) * b, b)]],
                             rows.at[0], sem_a)
            pltpu.sync_copy(rows.at[1], h_sh.at[did2.at[c1]], add=True)
            return carry

        lax.fori_loop(0, (nch - 1) // 2, step, 0)
        pltpu.make_async_copy(y_hbm.at[gid.at[pl.ds((nch - 1) * b, b)]],
                              rows.at[0], sem_a).wait()
        pltpu.sync_copy(rows.at[0], h_sh.at[did2.at[nch - 1]], add=True)

        plsc.subcore_barrier()

        @pl.when(sid < NS - 1)
        def _():
            pltpu.sync_copy(h_sh.at[pl.ds(sid * rpt_a, rpt_a)],
                            out_hbm.at[cid, pl.ds(sid * rpt_a, rpt_a)])

        @pl.when(sid == NS - 1)
        def _():
            pltpu.sync_copy(h_sh.at[pl.ds(rpt_a * (NS - 1), rpt_z)],
                            out_hbm.at[cid, pl.ds(rpt_a * (NS - 1), rpt_z)])

    return sc_scatter, ept, ept_p


def _pad_per_tile(a, ept, ept_p, fill):
    if ept_p == ept:
        return a
    pad = jnp.full((NW, ept_p - ept), fill, a.dtype)
    return jnp.concatenate([a.reshape(NW, ept), pad], axis=1).reshape(-1)


def kernel(x, edge_index, rel_type, weight):
    n, _ = x.shape
    r, _, d_out = weight.shape
    e = edge_index.shape[1]
    src = edge_index[0]
    dst = edge_index[1]
    y, g2 = _relation_gemm_gid(x, weight, src, rel_type, 1000)
    y = y.reshape(r * n, d_out)
    g = g2.reshape(e)
    sc, ept, ept_p = _make_sc_scatter(n, d_out, e)
    g_p = _pad_per_tile(g, ept, ept_p, 0)
    dst_p = _pad_per_tile(dst, ept, ept_p, n)
    zeros = jnp.zeros((n, d_out), jnp.float32)
    partials = sc(y, g_p, dst_p, zeros)
    return _pair_add(partials, 1000)


# speedup vs baseline: 1.3982x; 1.3259x over previous
import functools

import jax
import jax.numpy as jnp
from jax import lax
from jax.experimental import pallas as pl
from jax.experimental.pallas import tpu as pltpu
from jax.experimental.pallas import tpu_sc as plsc

NC = 2
NS = 16
NW = NC * NS


def _gemm_gid_body(n_nodes, x_ref, w_ref, s_ref, r_ref, o_ref, g_ref):
    o_ref[0] = jnp.dot(x_ref[...], w_ref[0],
                       preferred_element_type=jnp.float32)
    g_ref[...] = r_ref[...] * n_nodes + s_ref[...]


def _relation_gemm_gid(x, weight, src, rel, bn):
    n, d_in = x.shape
    r, _, d_out = weight.shape
    e = src.shape[0]
    nb = r * (n // bn)
    eb = 8
    ew = e // (nb * eb)
    assert eb * ew * nb == e
    s2 = src.reshape(nb * eb, ew)
    r2 = rel.reshape(nb * eb, ew)
    return pl.pallas_call(
        functools.partial(_gemm_gid_body, n),
        grid=(r, n // bn),
        in_specs=[
            pl.BlockSpec((bn, d_in), lambda i, j: (j, 0)),
            pl.BlockSpec((1, d_in, d_out), lambda i, j: (i, 0, 0)),
            pl.BlockSpec((eb, ew), lambda i, j, _nbj=n // bn: (i * _nbj + j, 0)),
            pl.BlockSpec((eb, ew), lambda i, j, _nbj=n // bn: (i * _nbj + j, 0)),
        ],
        out_specs=[
            pl.BlockSpec((1, bn, d_out), lambda i, j: (i, j, 0)),
            pl.BlockSpec((eb, ew), lambda i, j, _nbj=n // bn: (i * _nbj + j, 0)),
        ],
        out_shape=[
            jax.ShapeDtypeStruct((r, n, d_out), jnp.float32),
            jax.ShapeDtypeStruct((nb * eb, ew), jnp.int32),
        ],
    )(x, weight, s2, r2)


def _add_body(p_ref, o_ref):
    o_ref[...] = p_ref[0] + p_ref[1]


def _pair_add(p, bn):
    _, n, d = p.shape
    return pl.pallas_call(
        _add_body,
        grid=(n // bn,),
        in_specs=[pl.BlockSpec((2, bn, d), lambda i: (0, i, 0))],
        out_specs=pl.BlockSpec((bn, d), lambda i: (i, 0)),
        out_shape=jax.ShapeDtypeStruct((n, d), jnp.float32),
    )(p)


def _make_sc_scatter(n_nodes, d, n_edges):
    ept = n_edges // NW
    b = 80
    nch = -(-ept // b)
    nch += (1 - nch) % 2
    ept_p = nch * b
    rpt_a = (n_nodes // NS) & ~7
    rpt_z = n_nodes - rpt_a * (NS - 1)
    assert ept * NW == n_edges and nch % 2 == 1 and nch >= 5
    assert b % 8 == 0 and ept_p % 8 == 0 and rpt_a % 8 == 0

    mesh = plsc.VectorSubcoreMesh(core_axis_name="c", subcore_axis_name="s",
                                  num_cores=NC, num_subcores=NS)

    @functools.partial(
        pl.kernel,
        out_type=jax.ShapeDtypeStruct((NC, n_nodes, d), jnp.float32),
        mesh=mesh,
        scratch_types=[
            pltpu.VMEM((ept_p,), jnp.int32),
            pltpu.VMEM((nch, b), jnp.int32),
            pltpu.VMEM((2, b, d), jnp.float32),
            pltpu.VMEM_SHARED((n_nodes + 8, d), jnp.float32),
            pltpu.SemaphoreType.DMA,
            pltpu.SemaphoreType.DMA,
            pltpu.SemaphoreType.DMA,
            pltpu.SemaphoreType.DMA,
        ],
    )
    def sc_scatter(y_hbm, g_hbm, dst_hbm, zeros_hbm, out_hbm,
                   gid, did2, rows, h_sh, sem_i, sem_d, sem_a, sem_b):
        cid = lax.axis_index("c")
        sid = lax.axis_index("s")
        wid = cid * NS + sid
        base = wid * ept_p

        cp_g = pltpu.async_copy(g_hbm.at[pl.ds(base, ept_p)], gid, sem_i)

        def fill_did(c, carry):
            pltpu.async_copy(dst_hbm.at[pl.ds(base + c * b, b)],
                             did2.at[c], sem_d)
            return carry

        lax.fori_loop(0, nch, fill_did, 0)

        @pl.when(sid < NS - 1)
        def _():
            pltpu.sync_copy(zeros_hbm.at[pl.ds(sid * rpt_a, rpt_a)],
                            h_sh.at[pl.ds(sid * rpt_a, rpt_a)])

        @pl.when(sid == NS - 1)
        def _():
            pltpu.sync_copy(zeros_hbm.at[pl.ds(rpt_a * (NS - 1), rpt_z)],
                            h_sh.at[pl.ds(rpt_a * (NS - 1), rpt_z)])

        cp_g.wait()

        def drain_did(c, carry):
            pltpu.make_async_copy(dst_hbm.at[pl.ds(base, b)],
                                  did2.at[0], sem_d).wait()
            return carry

        lax.fori_loop(0, nch, drain_did, 0)

        plsc.subcore_barrier()

        pltpu.async_copy(y_hbm.at[gid.at[pl.ds(0, b)]], rows.at[0], sem_a)

        def step(t, carry):
            c0 = 2 * t
            c1 = 2 * t + 1
            pltpu.make_async_copy(y_hbm.at[gid.at[pl.ds(c0 * b, b)]],
                                  rows.at[0], sem_a).wait()
            pltpu.async_copy(y_hbm.at[gid.at[pl.ds(c1 * b, b)]],
                             rows.at[1], sem_b)
            pltpu.sync_copy(rows.at[0], h_sh.at[did2.at[c0]], add=True)
            pltpu.make_async_copy(y_hbm.at[gid.at[pl.ds(c1 * b, b)]],
                                  rows.at[1], sem_b).wait()
            pltpu.async_copy(y_hbm.at[gid.at[pl.ds((c1 + 1) * b, b)]],
                             rows.at[0], sem_a)
            pltpu.sync_copy(rows.at[1], h_sh.at[did2.at[c1]], add=True)
            return carry

        lax.fori_loop(0, (nch - 1) // 2, step, 0)
        pltpu.make_async_copy(y_hbm.at[gid.at[pl.ds((nch - 1) * b, b)]],
                              rows.at[0], sem_a).wait()
        pltpu.sync_copy(rows.at[0], h_sh.at[did2.at[nch - 1]], add=True)

        plsc.subcore_barrier()

        @pl.when(sid < NS - 1)
        def _():
            pltpu.sync_copy(h_sh.at[pl.ds(sid * rpt_a, rpt_a)],
                            out_hbm.at[cid, pl.ds(sid * rpt_a, rpt_a)])

        @pl.when(sid == NS - 1)
        def _():
            pltpu.sync_copy(h_sh.at[pl.ds(rpt_a * (NS - 1), rpt_z)],
                            out_hbm.at[cid, pl.ds(rpt_a * (NS - 1), rpt_z)])

    return sc_scatter, ept, ept_p


def _pad_per_tile(a, ept, ept_p, fill):
    if ept_p == ept:
        return a
    pad = jnp.full((NW, ept_p - ept), fill, a.dtype)
    return jnp.concatenate([a.reshape(NW, ept), pad], axis=1).reshape(-1)


def kernel(x, edge_index, rel_type, weight):
    n, _ = x.shape
    r, _, d_out = weight.shape
    e = edge_index.shape[1]
    src = edge_index[0]
    dst = edge_index[1]
    y, g2 = _relation_gemm_gid(x, weight, src, rel_type, 1000)
    y = y.reshape(r * n, d_out)
    g = g2.reshape(e)
    sc, ept, ept_p = _make_sc_scatter(n, d_out, e)
    g_p = _pad_per_tile(g, ept, ept_p, 0)
    dst_p = _pad_per_tile(dst, ept, ept_p, n)
    zeros = jnp.zeros((n, d_out), jnp.float32)
    partials = sc(y, g_p, dst_p, zeros)
    return _pair_add(partials, 1000)
